# baseline (speedup 1.0000x reference)
import functools

import jax
import jax.numpy as jnp
from jax import lax
from jax.experimental import pallas as pl
from jax.experimental.pallas import tpu as pltpu

N_NODES = 10000
N_EDGES = 320000
D = 128
DE = 16

NBLK = 1000
EBLK = 1000


def _silu(x):
    return x * jax.nn.sigmoid(x)


def _dot_t(x, w):
    return lax.dot_general(x.astype(jnp.bfloat16), w.astype(jnp.bfloat16),
                           (((1,), (1,)), ((), ())),
                           preferred_element_type=jnp.float32)


def _bf(x):
    return x.astype(jnp.bfloat16).astype(jnp.float32)


def _pre_body(h_ref, a1_ref, b1_ref, ha_ref, hb_ref):
    h = h_ref[...]
    ha_ref[...] = _dot_t(h, a1_ref[...])
    hb_ref[...] = _dot_t(h, b1_ref[...])


def _tc_pre(h, A1, B1):
    grid = (N_NODES // NBLK,)
    blk = pl.BlockSpec((NBLK, D), lambda i: (i, 0))
    wblk = pl.BlockSpec((D, D), lambda i: (0, 0))
    return pl.pallas_call(
        _pre_body,
        grid=grid,
        in_specs=[blk, wblk, wblk],
        out_specs=[blk, blk],
        out_shape=[jax.ShapeDtypeStruct((N_NODES, D), jnp.float32)] * 2,
    )(h, A1, B1)


def _edge_body(har_ref, hbc_ref, ea_ref, cdr_ref,
               c1t_ref, wr_ref, c1_ref, w2_ref, b2_ref, ae1_ref, cae_ref,
               ae2_ref, ef_ref, tr_ref):
    cdr = cdr_ref[...]
    rad = cdr[:, 3:4]
    ea = ea_ref[...]
    c1t = c1t_ref[...]
    t_ea = _bf(ea[:, 0:1]) * _bf(c1t[0:1, :])
    for k in range(1, DE):
        t_ea = t_ea + _bf(ea[:, k:k + 1]) * _bf(c1t[k:k + 1, :])
    t1 = (har_ref[...] + hbc_ref[...] + t_ea
          + _bf(rad) * wr_ref[...] + c1_ref[...])
    t1 = _silu(t1)
    ef = _silu(_dot_t(t1, w2_ref[...]) + b2_ref[...])
    g = _silu(_dot_t(ef, ae1_ref[...]) + cae_ref[...])
    ae = jnp.sum(_bf(g) * _bf(ae2_ref[...]), axis=1, keepdims=True)
    inv = 1.0 / jnp.maximum(jnp.sqrt(rad), 1e-12)
    ef_ref[...] = ef
    tr_ref[...] = cdr * (ae * inv)


def _tc_edge(har, hbc, ea, cdr, C1, wr, c1, W2, b2, Ae, cae, Ae2):
    grid = (N_EDGES // EBLK,)
    eblk = pl.BlockSpec((EBLK, D), lambda i: (i, 0))
    eablk = pl.BlockSpec((EBLK, DE), lambda i: (i, 0))
    cdblk = pl.BlockSpec((EBLK, 4), lambda i: (i, 0))
    full = lambda shape: pl.BlockSpec(shape, lambda i: (0,) * len(shape))
    return pl.pallas_call(
        _edge_body,
        grid=grid,
        in_specs=[eblk, eblk, eablk, cdblk,
                  full((DE, D)), full((1, D)), full((1, D)),
                  full((D, D)), full((1, D)), full((D, D)), full((1, D)),
                  full((1, D))],
        out_specs=[eblk, cdblk],
        out_shape=[jax.ShapeDtypeStruct((N_EDGES, D), jnp.float32),
                   jax.ShapeDtypeStruct((N_EDGES, 4), jnp.float32)],
    )(har, hbc, ea, cdr, C1, wr, c1, W2, b2, Ae, cae, Ae2)


def _node_body(h_ref, ag0_ref, ag1_ref, ac0_ref, ac1_ref,
               nh_ref, na_ref, cn1_ref, n2_ref, bn2_ref,
               ph_ref, cap_ref, ap2_ref, hnew_ref, acc_ref):
    h = h_ref[...]
    agg = ag0_ref[...] + ag1_ref[...]
    hn = _silu(_dot_t(h, nh_ref[...]) + _dot_t(agg, na_ref[...]) + cn1_ref[...])
    hn = _silu(_dot_t(hn, n2_ref[...]) + bn2_ref[...])
    h_new = h + hn
    p = _silu(_dot_t(h_new, ph_ref[...]) + cap_ref[...])
    ap = jnp.sum(_bf(p) * _bf(ap2_ref[...]), axis=1, keepdims=True)
    aggc = ac0_ref[...] + ac1_ref[...]
    hnew_ref[...] = h_new
    acc_ref[...] = aggc * ap


def _tc_node(h, ag0, ag1, ac0, ac1, Nh, Na, cn1, W_n2, b_n2, Ph, cap, Ap2):
    grid = (N_NODES // NBLK,)
    blk = pl.BlockSpec((NBLK, D), lambda i: (i, 0))
    cblk = pl.BlockSpec((NBLK, 4), lambda i: (i, 0))
    full = lambda shape: pl.BlockSpec(shape, lambda i: (0,) * len(shape))
    return pl.pallas_call(
        _node_body,
        grid=grid,
        in_specs=[blk, blk, blk, cblk, cblk,
                  full((D, D)), full((D, D)), full((1, D)),
                  full((D, D)), full((1, D)), full((D, D)), full((1, D)),
                  full((1, D))],
        out_specs=[blk, cblk],
        out_shape=[jax.ShapeDtypeStruct((N_NODES, D), jnp.float32),
                   jax.ShapeDtypeStruct((N_NODES, 4), jnp.float32)],
    )(h, ag0, ag1, ac0, ac1, Nh, Na, cn1, W_n2, b_n2, Ph, cap, Ap2)


def kernel(h, edge_index, coord, edge_attr, prompt, W_e1, b_e1, W_e2, b_e2,
           W_n1, b_n1, W_n2, b_n2, W_ae1, b_ae1, W_ae2, W_ap1, b_ap1, W_ap2):
    row = edge_index[0].astype(jnp.int32)
    col = edge_index[1].astype(jnp.int32)
    p0 = prompt[0]

    def _fold(w, b):
        return (jnp.matmul(w.astype(jnp.bfloat16), p0.astype(jnp.bfloat16),
                           preferred_element_type=jnp.float32) + b)[None, :]

    A1 = W_e1[:, 0:D]
    B1 = W_e1[:, D:2 * D]
    wr = W_e1[:, 2 * D].astype(jnp.bfloat16).astype(jnp.float32)[None, :]
    C1 = W_e1[:, 2 * D + 1:2 * D + 1 + DE]
    c1 = _fold(W_e1[:, 2 * D + 1 + DE:], b_e1)
    cae = _fold(W_ae1[:, D:], b_ae1)
    Ae = W_ae1[:, :D]
    cn1 = _fold(W_n1[:, 2 * D:], b_n1)
    Nh = W_n1[:, 0:D]
    Na = W_n1[:, D:2 * D]
    cap = _fold(W_ap1[:, D:], b_ap1)
    Ph = W_ap1[:, :D]

    hA, hB = _tc_pre(h, A1, B1)

    har = hA[row]
    hbc = hB[col]
    cd3 = coord[row] - coord[col]
    radial = jnp.sum(cd3 * cd3, axis=1, keepdims=True)
    cdr = jnp.concatenate([cd3, radial], axis=1)

    ef, tr = _tc_edge(har, hbc, edge_attr, cdr, C1.T, wr, c1,
                      W_e2, b_e2[None, :], Ae, cae, W_ae2)

    half = N_EDGES // 2
    ag0 = jnp.zeros((N_NODES, D), jnp.float32).at[row[:half]].add(ef[:half])
    ag1 = jnp.zeros((N_NODES, D), jnp.float32).at[row[half:]].add(ef[half:])
    ac0 = jnp.zeros((N_NODES, 4), jnp.float32).at[row[:half]].add(tr[:half])
    ac1 = jnp.zeros((N_NODES, 4), jnp.float32).at[row[half:]].add(tr[half:])

    h_new, acc4 = _tc_node(h, ag0, ag1, ac0, ac1, Nh, Na, cn1,
                           W_n2, b_n2[None, :], Ph, cap, W_ap2)
    return (h_new, coord, acc4[:, :3])


# speedup vs baseline: 2.8595x; 2.8595x over previous
import functools

import jax
import jax.numpy as jnp
from jax import lax
from jax.experimental import pallas as pl
from jax.experimental.pallas import tpu as pltpu
from jax.experimental.pallas import tpu_sc as plsc

N_NODES = 10000
N_EDGES = 320000
D = 128
DE = 16

NBLK = 1000
EBLK = 1000

NC = 2
NS = 16
NW = NC * NS
EPW = N_EDGES // NW
CG = 80
NP = 10112
NSL = NP // NS


def _silu(x):
    return x * jax.nn.sigmoid(x)


def _dot_t(x, w):
    return lax.dot_general(x.astype(jnp.bfloat16), w.astype(jnp.bfloat16),
                           (((1,), (1,)), ((), ())),
                           preferred_element_type=jnp.float32)


def _bf(x):
    return x.astype(jnp.bfloat16).astype(jnp.float32)


def _pre_body(h_ref, a1_ref, b1_ref, ha_ref, hb_ref):
    h = h_ref[...]
    ha_ref[...] = _dot_t(h, a1_ref[...])
    hb_ref[...] = _dot_t(h, b1_ref[...])


def _tc_pre(h, A1, B1):
    grid = (N_NODES // NBLK,)
    blk = pl.BlockSpec((NBLK, D), lambda i: (i, 0))
    wblk = pl.BlockSpec((D, D), lambda i: (0, 0))
    return pl.pallas_call(
        _pre_body,
        grid=grid,
        in_specs=[blk, wblk, wblk],
        out_specs=[blk, blk],
        out_shape=[jax.ShapeDtypeStruct((N_NODES, D), jnp.float32)] * 2,
    )(h, A1, B1)


def _edge_body(har_ref, hbc_ref, ea_ref, cdr_ref,
               c1t_ref, wr_ref, c1_ref, w2_ref, b2_ref, ae1_ref, cae_ref,
               ae2_ref, ef_ref, tr_ref):
    cdr = cdr_ref[...]
    rad = cdr[:, 3:4]
    ea = ea_ref[...]
    c1t = c1t_ref[...]
    t_ea = _bf(ea[:, 0:1]) * _bf(c1t[0:1, :])
    for k in range(1, DE):
        t_ea = t_ea + _bf(ea[:, k:k + 1]) * _bf(c1t[k:k + 1, :])
    t1 = (har_ref[...] + hbc_ref[...] + t_ea
          + _bf(rad) * wr_ref[...] + c1_ref[...])
    t1 = _silu(t1)
    ef = _silu(_dot_t(t1, w2_ref[...]) + b2_ref[...])
    g = _silu(_dot_t(ef, ae1_ref[...]) + cae_ref[...])
    ae = jnp.sum(_bf(g) * _bf(ae2_ref[...]), axis=1, keepdims=True)
    inv = 1.0 / jnp.maximum(jnp.sqrt(rad), 1e-12)
    ef_ref[...] = ef
    tr4 = cdr * (ae * inv)
    tr_ref[...] = jnp.concatenate(
        [tr4, jnp.zeros((tr4.shape[0], D - 4), jnp.float32)], axis=1)


def _tc_edge(har, hbc, ea, cdr, C1, wr, c1, W2, b2, Ae, cae, Ae2):
    grid = (N_EDGES // EBLK,)
    eblk = pl.BlockSpec((EBLK, D), lambda i: (i, 0))
    eablk = pl.BlockSpec((EBLK, DE), lambda i: (i, 0))
    cdblk = pl.BlockSpec((EBLK, 4), lambda i: (i, 0))
    full = lambda shape: pl.BlockSpec(shape, lambda i: (0,) * len(shape))
    return pl.pallas_call(
        _edge_body,
        grid=grid,
        in_specs=[eblk, eblk, eablk, cdblk,
                  full((DE, D)), full((1, D)), full((1, D)),
                  full((D, D)), full((1, D)), full((D, D)), full((1, D)),
                  full((1, D))],
        out_specs=[eblk, eblk],
        out_shape=[jax.ShapeDtypeStruct((N_EDGES, D), jnp.float32),
                   jax.ShapeDtypeStruct((N_EDGES, D), jnp.float32)],
    )(har, hbc, ea, cdr, C1, wr, c1, W2, b2, Ae, cae, Ae2)


def _node_body(h_ref, ag0_ref, ag1_ref, ac0_ref, ac1_ref,
               nh_ref, na_ref, cn1_ref, n2_ref, bn2_ref,
               ph_ref, cap_ref, ap2_ref, hnew_ref, acc_ref):
    h = h_ref[...]
    agg = ag0_ref[...] + ag1_ref[...]
    hn = _silu(_dot_t(h, nh_ref[...]) + _dot_t(agg, na_ref[...]) + cn1_ref[...])
    hn = _silu(_dot_t(hn, n2_ref[...]) + bn2_ref[...])
    h_new = h + hn
    p = _silu(_dot_t(h_new, ph_ref[...]) + cap_ref[...])
    ap = jnp.sum(_bf(p) * _bf(ap2_ref[...]), axis=1, keepdims=True)
    aggc = ac0_ref[...] + ac1_ref[...]
    hnew_ref[...] = h_new
    acc_ref[...] = aggc * ap


def _tc_node(h, ag0, ag1, ac0, ac1, Nh, Na, cn1, W_n2, b_n2, Ph, cap, Ap2):
    grid = (N_NODES // NBLK,)
    blk = pl.BlockSpec((NBLK, D), lambda i: (i, 0))
    cblk = pl.BlockSpec((NBLK, 4), lambda i: (i, 0))
    full = lambda shape: pl.BlockSpec(shape, lambda i: (0,) * len(shape))
    return pl.pallas_call(
        _node_body,
        grid=grid,
        in_specs=[blk, blk, blk, cblk, cblk,
                  full((D, D)), full((D, D)), full((1, D)),
                  full((D, D)), full((1, D)), full((D, D)), full((1, D)),
                  full((1, D))],
        out_specs=[blk, cblk],
        out_shape=[jax.ShapeDtypeStruct((N_NODES, D), jnp.float32),
                   jax.ShapeDtypeStruct((N_NODES, 4), jnp.float32)],
    )(h, ag0, ag1, ac0, ac1, Nh, Na, cn1, W_n2, b_n2, Ph, cap, Ap2)


def _sc_gather_body(hA, hB, rowh, colh, cfl, outA, outB, outC,
                    ct_v, ri_v, ci_v, rows_v, cdr_v, sem):
    cid = lax.axis_index("c")
    sid = lax.axis_index("s")
    wid = sid * NC + cid
    pltpu.sync_copy(cfl, ct_v)

    def chunk(j, carry):
        base = wid * EPW + j * CG
        pltpu.sync_copy(rowh.at[pl.ds(base, CG)], ri_v)
        pltpu.sync_copy(colh.at[pl.ds(base, CG)], ci_v)
        pltpu.async_copy(hA.at[ri_v], rows_v, sem).wait()
        pltpu.sync_copy(rows_v, outA.at[pl.ds(base, CG)])
        pltpu.async_copy(hB.at[ci_v], rows_v, sem).wait()
        pltpu.sync_copy(rows_v, outB.at[pl.ds(base, CG)])
        for i in range(CG // 16):
            ri3 = ri_v[pl.ds(i * 16, 16)] * 3
            ci3 = ci_v[pl.ds(i * 16, 16)] * 3
            lanes = lax.iota(jnp.int32, 16) + (i * 16)
            rad = None
            for k in range(3):
                a = plsc.load_gather(ct_v, [ri3 + k])
                b = plsc.load_gather(ct_v, [ci3 + k])
                dk = a - b
                plsc.store_scatter(cdr_v, [lanes, jnp.full((16,), k, jnp.int32)], dk)
                rad = dk * dk if rad is None else rad + dk * dk
            plsc.store_scatter(cdr_v, [lanes, jnp.full((16,), 3, jnp.int32)], rad)
        pltpu.sync_copy(cdr_v, outC.at[pl.ds(base, CG)])
        return carry

    lax.fori_loop(0, EPW // CG, chunk, 0)


def _sc_gather(hA, hB, row, col, coord_flat):
    mesh = plsc.VectorSubcoreMesh(core_axis_name="c", subcore_axis_name="s")
    fn = pl.kernel(
        _sc_gather_body,
        out_type=[jax.ShapeDtypeStruct((N_EDGES, D), jnp.float32),
                  jax.ShapeDtypeStruct((N_EDGES, D), jnp.float32),
                  jax.ShapeDtypeStruct((N_EDGES, 4), jnp.float32)],
        mesh=mesh,
        scratch_types=[pltpu.VMEM((3 * N_NODES,), jnp.float32),
                       pltpu.VMEM((CG,), jnp.int32),
                       pltpu.VMEM((CG,), jnp.int32),
                       pltpu.VMEM((CG, D), jnp.float32),
                       pltpu.VMEM((CG, 4), jnp.float32),
                       pltpu.SemaphoreType.DMA],
        compiler_params=pltpu.CompilerParams(needs_layout_passes=False),
    )
    return fn(hA, hB, row, col, coord_flat)


def _sc_scatter_body(vals, rowh, z128, out2, acc_sh, val_v, idx_v):
    cid = lax.axis_index("c")
    sid = lax.axis_index("s")
    pltpu.sync_copy(z128.at[pl.ds(sid * NSL, NSL)],
                    acc_sh.at[pl.ds(sid * NSL, NSL)])
    plsc.subcore_barrier()

    def chunk(j, carry):
        base = cid * (N_EDGES // NC) + sid * EPW + j * CG
        pltpu.sync_copy(rowh.at[pl.ds(base, CG)], idx_v)
        pltpu.sync_copy(vals.at[pl.ds(base, CG)], val_v)
        pltpu.sync_copy(val_v, acc_sh.at[idx_v], add=True)
        return carry

    lax.fori_loop(0, EPW // CG, chunk, 0)
    plsc.subcore_barrier()
    pltpu.sync_copy(acc_sh.at[pl.ds(sid * NSL, NSL)],
                    out2.at[pl.ds(cid * NP + sid * NSL, NSL)])


def _sc_scatter(vals, row, z128):
    mesh = plsc.VectorSubcoreMesh(core_axis_name="c", subcore_axis_name="s")
    fn = pl.kernel(
        _sc_scatter_body,
        out_type=[jax.ShapeDtypeStruct((2 * NP, D), jnp.float32)],
        mesh=mesh,
        scratch_types=[pltpu.VMEM_SHARED((NP, D), jnp.float32),
                       pltpu.VMEM((CG, D), jnp.float32),
                       pltpu.VMEM((CG,), jnp.int32)],
    )
    (out2,) = fn(vals, row, z128)
    return out2


def kernel(h, edge_index, coord, edge_attr, prompt, W_e1, b_e1, W_e2, b_e2,
           W_n1, b_n1, W_n2, b_n2, W_ae1, b_ae1, W_ae2, W_ap1, b_ap1, W_ap2):
    row = edge_index[0].astype(jnp.int32)
    col = edge_index[1].astype(jnp.int32)
    p0 = prompt[0]

    def _fold(w, b):
        return (jnp.matmul(w.astype(jnp.bfloat16), p0.astype(jnp.bfloat16),
                           preferred_element_type=jnp.float32) + b)[None, :]

    A1 = W_e1[:, 0:D]
    B1 = W_e1[:, D:2 * D]
    wr = W_e1[:, 2 * D].astype(jnp.bfloat16).astype(jnp.float32)[None, :]
    C1 = W_e1[:, 2 * D + 1:2 * D + 1 + DE]
    c1 = _fold(W_e1[:, 2 * D + 1 + DE:], b_e1)
    cae = _fold(W_ae1[:, D:], b_ae1)
    Ae = W_ae1[:, :D]
    cn1 = _fold(W_n1[:, 2 * D:], b_n1)
    Nh = W_n1[:, 0:D]
    Na = W_n1[:, D:2 * D]
    cap = _fold(W_ap1[:, D:], b_ap1)
    Ph = W_ap1[:, :D]

    hA, hB = _tc_pre(h, A1, B1)

    har, hbc, cdr = _sc_gather(hA, hB, row, col, coord.reshape(-1))

    ef, tr = _tc_edge(har, hbc, edge_attr, cdr, C1.T, wr, c1,
                      W_e2, b_e2[None, :], Ae, cae, W_ae2)

    z128 = jnp.zeros((NP, D), jnp.float32)
    agg2 = _sc_scatter(ef, row, z128)
    acc2 = _sc_scatter(tr, row, z128)
    ag0, ag1 = agg2[:N_NODES], agg2[NP:NP + N_NODES]
    ac0, ac1 = acc2[:N_NODES, :4], acc2[NP:NP + N_NODES, :4]

    h_new, acc4 = _tc_node(h, ag0, ag1, ac0, ac1, Nh, Na, cn1,
                           W_n2, b_n2[None, :], Ph, cap, W_ap2)
    return (h_new, coord, acc4[:, :3])


# speedup vs baseline: 3.2437x; 1.1343x over previous
import functools

import jax
import jax.numpy as jnp
from jax import lax
from jax.experimental import pallas as pl
from jax.experimental.pallas import tpu as pltpu
from jax.experimental.pallas import tpu_sc as plsc

N_NODES = 10000
N_EDGES = 320000
D = 128
DE = 16

NBLK = 1000
EBLK = 1000

NC = 2
NS = 16
NW = NC * NS
EPW = N_EDGES // NW
CG = 80
NP = 10112
NSL = NP // NS


def _silu(x):
    return x * jax.nn.sigmoid(x)


def _dot_t(x, w):
    return lax.dot_general(x.astype(jnp.bfloat16), w.astype(jnp.bfloat16),
                           (((1,), (1,)), ((), ())),
                           preferred_element_type=jnp.float32)


def _bf(x):
    return x.astype(jnp.bfloat16).astype(jnp.float32)


def _pre_body(h_ref, a1_ref, b1_ref, ha_ref, hb_ref):
    h = h_ref[...]
    ha_ref[...] = _dot_t(h, a1_ref[...])
    hb_ref[...] = _dot_t(h, b1_ref[...])


def _tc_pre(h, A1, B1):
    grid = (N_NODES // NBLK,)
    blk = pl.BlockSpec((NBLK, D), lambda i: (i, 0))
    wblk = pl.BlockSpec((D, D), lambda i: (0, 0))
    return pl.pallas_call(
        _pre_body,
        grid=grid,
        in_specs=[blk, wblk, wblk],
        out_specs=[blk, blk],
        out_shape=[jax.ShapeDtypeStruct((N_NODES, D), jnp.float32)] * 2,
    )(h, A1, B1)


def _edge_body(har_ref, hbc_ref, ea_ref, cdr_ref,
               c1t_ref, wr_ref, c1_ref, w2_ref, b2_ref, ae1_ref, cae_ref,
               ae2_ref, ef_ref, tr_ref):
    cdr = cdr_ref[...]
    rad = cdr[:, 3:4]
    ea = ea_ref[...]
    c1t = c1t_ref[...]
    t_ea = _bf(ea[:, 0:1]) * _bf(c1t[0:1, :])
    for k in range(1, DE):
        t_ea = t_ea + _bf(ea[:, k:k + 1]) * _bf(c1t[k:k + 1, :])
    t1 = (har_ref[...] + hbc_ref[...] + t_ea
          + _bf(rad) * wr_ref[...] + c1_ref[...])
    t1 = _silu(t1)
    ef = _silu(_dot_t(t1, w2_ref[...]) + b2_ref[...])
    g = _silu(_dot_t(ef, ae1_ref[...]) + cae_ref[...])
    ae = jnp.sum(_bf(g) * _bf(ae2_ref[...]), axis=1, keepdims=True)
    inv = 1.0 / jnp.maximum(jnp.sqrt(rad), 1e-12)
    ef_ref[...] = ef
    tr4 = cdr * (ae * inv)
    tr_ref[...] = jnp.concatenate(
        [tr4, jnp.zeros((tr4.shape[0], D - 4), jnp.float32)], axis=1)


def _tc_edge(har, hbc, ea, cdr, C1, wr, c1, W2, b2, Ae, cae, Ae2):
    grid = (N_EDGES // EBLK,)
    eblk = pl.BlockSpec((EBLK, D), lambda i: (i, 0))
    eablk = pl.BlockSpec((EBLK, DE), lambda i: (i, 0))
    cdblk = pl.BlockSpec((EBLK, 4), lambda i: (i, 0))
    full = lambda shape: pl.BlockSpec(shape, lambda i: (0,) * len(shape))
    return pl.pallas_call(
        _edge_body,
        grid=grid,
        in_specs=[eblk, eblk, eablk, cdblk,
                  full((DE, D)), full((1, D)), full((1, D)),
                  full((D, D)), full((1, D)), full((D, D)), full((1, D)),
                  full((1, D))],
        out_specs=[eblk, eblk],
        out_shape=[jax.ShapeDtypeStruct((N_EDGES, D), jnp.float32),
                   jax.ShapeDtypeStruct((N_EDGES, D), jnp.float32)],
    )(har, hbc, ea, cdr, C1, wr, c1, W2, b2, Ae, cae, Ae2)


def _node_body(h_ref, ag0_ref, ag1_ref, ac0_ref, ac1_ref,
               nh_ref, na_ref, cn1_ref, n2_ref, bn2_ref,
               ph_ref, cap_ref, ap2_ref, hnew_ref, acc_ref):
    h = h_ref[...]
    agg = ag0_ref[...] + ag1_ref[...]
    hn = _silu(_dot_t(h, nh_ref[...]) + _dot_t(agg, na_ref[...]) + cn1_ref[...])
    hn = _silu(_dot_t(hn, n2_ref[...]) + bn2_ref[...])
    h_new = h + hn
    p = _silu(_dot_t(h_new, ph_ref[...]) + cap_ref[...])
    ap = jnp.sum(_bf(p) * _bf(ap2_ref[...]), axis=1, keepdims=True)
    aggc = ac0_ref[...] + ac1_ref[...]
    hnew_ref[...] = h_new
    acc_ref[...] = aggc * ap


def _tc_node(h, ag0, ag1, ac0, ac1, Nh, Na, cn1, W_n2, b_n2, Ph, cap, Ap2):
    grid = (N_NODES // NBLK,)
    blk = pl.BlockSpec((NBLK, D), lambda i: (i, 0))
    cblk = pl.BlockSpec((NBLK, 4), lambda i: (i, 0))
    full = lambda shape: pl.BlockSpec(shape, lambda i: (0,) * len(shape))
    return pl.pallas_call(
        _node_body,
        grid=grid,
        in_specs=[blk, blk, blk, cblk, cblk,
                  full((D, D)), full((D, D)), full((1, D)),
                  full((D, D)), full((1, D)), full((D, D)), full((1, D)),
                  full((1, D))],
        out_specs=[blk, cblk],
        out_shape=[jax.ShapeDtypeStruct((N_NODES, D), jnp.float32),
                   jax.ShapeDtypeStruct((N_NODES, 4), jnp.float32)],
    )(h, ag0, ag1, ac0, ac1, Nh, Na, cn1, W_n2, b_n2, Ph, cap, Ap2)


CB = 128
NBIG = EPW // CB
CT = EPW - NBIG * CB


def _sc_gather_body(hA, hB, rowh, colh, cfl, outA, outB, outC,
                    ct_v, ri_v, ci_v, rA_v, rB_v, cdr_v,
                    ti_v, tj_v, tA_v, tB_v, tc_v,
                    semA, semB, semWA, semWB):
    cid = lax.axis_index("c")
    sid = lax.axis_index("s")
    wid = sid * NC + cid
    pltpu.sync_copy(cfl, ct_v)

    def coord_groups(riv, civ, cdrv, ngroups):
        for i in range(ngroups):
            ri3 = riv[pl.ds(i * 16, 16)] * 3
            ci3 = civ[pl.ds(i * 16, 16)] * 3
            lanes = lax.iota(jnp.int32, 16) + (i * 16)
            rad = None
            for k in range(3):
                a = plsc.load_gather(ct_v, [ri3 + k])
                b = plsc.load_gather(ct_v, [ci3 + k])
                dk = a - b
                plsc.store_scatter(cdrv, [lanes, jnp.full((16,), k, jnp.int32)], dk)
                rad = dk * dk if rad is None else rad + dk * dk
            plsc.store_scatter(cdrv, [lanes, jnp.full((16,), 3, jnp.int32)], rad)

    def do_chunk(base, n, riv, civ, rA, rB, cdrv):
        pltpu.sync_copy(rowh.at[pl.ds(base, n)], riv)
        pltpu.sync_copy(colh.at[pl.ds(base, n)], civ)
        gA = pltpu.async_copy(hA.at[riv], rA, semA)
        gB = pltpu.async_copy(hB.at[civ], rB, semB)
        gA.wait()
        wA = pltpu.async_copy(rA, outA.at[pl.ds(base, n)], semWA)
        gB.wait()
        wB = pltpu.async_copy(rB, outB.at[pl.ds(base, n)], semWB)
        coord_groups(riv, civ, cdrv, n // 16)
        pltpu.sync_copy(cdrv, outC.at[pl.ds(base, n)])
        wA.wait()
        wB.wait()

    def chunk(j, carry):
        do_chunk(wid * EPW + j * CB, CB, ri_v, ci_v, rA_v, rB_v, cdr_v)
        return carry

    lax.fori_loop(0, NBIG, chunk, 0)
    do_chunk(wid * EPW + NBIG * CB, CT, ti_v, tj_v, tA_v, tB_v, tc_v)


def _sc_gather(hA, hB, row, col, coord_flat):
    mesh = plsc.VectorSubcoreMesh(core_axis_name="c", subcore_axis_name="s")
    fn = pl.kernel(
        _sc_gather_body,
        out_type=[jax.ShapeDtypeStruct((N_EDGES, D), jnp.float32),
                  jax.ShapeDtypeStruct((N_EDGES, D), jnp.float32),
                  jax.ShapeDtypeStruct((N_EDGES, 4), jnp.float32)],
        mesh=mesh,
        scratch_types=[pltpu.VMEM((3 * N_NODES,), jnp.float32),
                       pltpu.VMEM((CB,), jnp.int32),
                       pltpu.VMEM((CB,), jnp.int32),
                       pltpu.VMEM((CB, D), jnp.float32),
                       pltpu.VMEM((CB, D), jnp.float32),
                       pltpu.VMEM((CB, 4), jnp.float32),
                       pltpu.VMEM((CT,), jnp.int32),
                       pltpu.VMEM((CT,), jnp.int32),
                       pltpu.VMEM((CT, D), jnp.float32),
                       pltpu.VMEM((CT, D), jnp.float32),
                       pltpu.VMEM((CT, 4), jnp.float32),
                       pltpu.SemaphoreType.DMA,
                       pltpu.SemaphoreType.DMA,
                       pltpu.SemaphoreType.DMA,
                       pltpu.SemaphoreType.DMA],
        compiler_params=pltpu.CompilerParams(needs_layout_passes=False),
    )
    return fn(hA, hB, row, col, coord_flat)


def _sc_scatter_body(vals, rowh, z128, out2, acc_sh, val_v, idx_v):
    cid = lax.axis_index("c")
    sid = lax.axis_index("s")
    pltpu.sync_copy(z128.at[pl.ds(sid * NSL, NSL)],
                    acc_sh.at[pl.ds(sid * NSL, NSL)])
    plsc.subcore_barrier()

    def chunk(j, carry):
        base = cid * (N_EDGES // NC) + sid * EPW + j * CG
        pltpu.sync_copy(rowh.at[pl.ds(base, CG)], idx_v)
        pltpu.sync_copy(vals.at[pl.ds(base, CG)], val_v)
        pltpu.sync_copy(val_v, acc_sh.at[idx_v], add=True)
        return carry

    lax.fori_loop(0, EPW // CG, chunk, 0)
    plsc.subcore_barrier()
    pltpu.sync_copy(acc_sh.at[pl.ds(sid * NSL, NSL)],
                    out2.at[pl.ds(cid * NP + sid * NSL, NSL)])


def _sc_scatter(vals, row, z128):
    mesh = plsc.VectorSubcoreMesh(core_axis_name="c", subcore_axis_name="s")
    fn = pl.kernel(
        _sc_scatter_body,
        out_type=[jax.ShapeDtypeStruct((2 * NP, D), jnp.float32)],
        mesh=mesh,
        scratch_types=[pltpu.VMEM_SHARED((NP, D), jnp.float32),
                       pltpu.VMEM((CG, D), jnp.float32),
                       pltpu.VMEM((CG,), jnp.int32)],
    )
    (out2,) = fn(vals, row, z128)
    return out2


def kernel(h, edge_index, coord, edge_attr, prompt, W_e1, b_e1, W_e2, b_e2,
           W_n1, b_n1, W_n2, b_n2, W_ae1, b_ae1, W_ae2, W_ap1, b_ap1, W_ap2):
    row = edge_index[0].astype(jnp.int32)
    col = edge_index[1].astype(jnp.int32)
    p0 = prompt[0]

    def _fold(w, b):
        return (jnp.matmul(w.astype(jnp.bfloat16), p0.astype(jnp.bfloat16),
                           preferred_element_type=jnp.float32) + b)[None, :]

    A1 = W_e1[:, 0:D]
    B1 = W_e1[:, D:2 * D]
    wr = W_e1[:, 2 * D].astype(jnp.bfloat16).astype(jnp.float32)[None, :]
    C1 = W_e1[:, 2 * D + 1:2 * D + 1 + DE]
    c1 = _fold(W_e1[:, 2 * D + 1 + DE:], b_e1)
    cae = _fold(W_ae1[:, D:], b_ae1)
    Ae = W_ae1[:, :D]
    cn1 = _fold(W_n1[:, 2 * D:], b_n1)
    Nh = W_n1[:, 0:D]
    Na = W_n1[:, D:2 * D]
    cap = _fold(W_ap1[:, D:], b_ap1)
    Ph = W_ap1[:, :D]

    hA, hB = _tc_pre(h, A1, B1)

    har, hbc, cdr = _sc_gather(hA, hB, row, col, coord.reshape(-1))

    ef, tr = _tc_edge(har, hbc, edge_attr, cdr, C1.T, wr, c1,
                      W_e2, b_e2[None, :], Ae, cae, W_ae2)

    z128 = jnp.zeros((NP, D), jnp.float32)
    agg2 = _sc_scatter(ef, row, z128)
    acc2 = _sc_scatter(tr, row, z128)
    ag0, ag1 = agg2[:N_NODES], agg2[NP:NP + N_NODES]
    ac0, ac1 = acc2[:N_NODES, :4], acc2[NP:NP + N_NODES, :4]

    h_new, acc4 = _tc_node(h, ag0, ag1, ac0, ac1, Nh, Na, cn1,
                           W_n2, b_n2[None, :], Ph, cap, W_ap2)
    return (h_new, coord, acc4[:, :3])


# speedup vs baseline: 3.5892x; 1.1065x over previous
import functools

import jax
import jax.numpy as jnp
from jax import lax
from jax.experimental import pallas as pl
from jax.experimental.pallas import tpu as pltpu
from jax.experimental.pallas import tpu_sc as plsc

N_NODES = 10000
N_EDGES = 320000
D = 128
DE = 16

NBLK = 1000
EBLK = 1000

NC = 2
NS = 16
NW = NC * NS
EPW = N_EDGES // NW
CG = 80
NP = 10112
NSL = NP // NS


def _silu(x):
    return x * jax.nn.sigmoid(x)


def _dot_t(x, w):
    return lax.dot_general(x.astype(jnp.bfloat16), w.astype(jnp.bfloat16),
                           (((1,), (1,)), ((), ())),
                           preferred_element_type=jnp.float32)


def _bf(x):
    return x.astype(jnp.bfloat16).astype(jnp.float32)


def _pre_body(h_ref, a1_ref, b1_ref, ha_ref, hb_ref):
    h = h_ref[...]
    ha_ref[...] = _dot_t(h, a1_ref[...])
    hb_ref[...] = _dot_t(h, b1_ref[...])


def _tc_pre(h, A1, B1):
    grid = (N_NODES // NBLK,)
    blk = pl.BlockSpec((NBLK, D), lambda i: (i, 0))
    wblk = pl.BlockSpec((D, D), lambda i: (0, 0))
    return pl.pallas_call(
        _pre_body,
        grid=grid,
        in_specs=[blk, wblk, wblk],
        out_specs=[blk, blk],
        out_shape=[jax.ShapeDtypeStruct((N_NODES, D), jnp.float32)] * 2,
    )(h, A1, B1)


def _edge_body(har_ref, hbc_ref, ea_ref, cdr_ref,
               c1t_ref, wr_ref, c1_ref, w2_ref, b2_ref, ae1_ref, cae_ref,
               ae2_ref, ef_ref, tr_ref):
    cdr = cdr_ref[...]
    rad = cdr[:, 3:4]
    ea = ea_ref[...]
    c1t = c1t_ref[...]
    t_ea = _bf(ea[:, 0:1]) * _bf(c1t[0:1, :])
    for k in range(1, DE):
        t_ea = t_ea + _bf(ea[:, k:k + 1]) * _bf(c1t[k:k + 1, :])
    t1 = (har_ref[...] + hbc_ref[...] + t_ea
          + _bf(rad) * wr_ref[...] + c1_ref[...])
    t1 = _silu(t1)
    ef = _silu(_dot_t(t1, w2_ref[...]) + b2_ref[...])
    g = _silu(_dot_t(ef, ae1_ref[...]) + cae_ref[...])
    ae = jnp.sum(_bf(g) * _bf(ae2_ref[...]), axis=1, keepdims=True)
    inv = 1.0 / jnp.maximum(jnp.sqrt(rad), 1e-12)
    ef_ref[...] = ef
    tr4 = cdr * (ae * inv)
    tr_ref[...] = jnp.concatenate(
        [tr4, jnp.zeros((tr4.shape[0], D - 4), jnp.float32)], axis=1)


def _tc_edge(har, hbc, ea, cdr, C1, wr, c1, W2, b2, Ae, cae, Ae2):
    grid = (N_EDGES // EBLK,)
    eblk = pl.BlockSpec((EBLK, D), lambda i: (i, 0))
    eablk = pl.BlockSpec((EBLK, DE), lambda i: (i, 0))
    cdblk = pl.BlockSpec((EBLK, 4), lambda i: (i, 0))
    full = lambda shape: pl.BlockSpec(shape, lambda i: (0,) * len(shape))
    return pl.pallas_call(
        _edge_body,
        grid=grid,
        in_specs=[eblk, eblk, eablk, cdblk,
                  full((DE, D)), full((1, D)), full((1, D)),
                  full((D, D)), full((1, D)), full((D, D)), full((1, D)),
                  full((1, D))],
        out_specs=[eblk, eblk],
        out_shape=[jax.ShapeDtypeStruct((N_EDGES, D), jnp.float32),
                   jax.ShapeDtypeStruct((N_EDGES, D), jnp.float32)],
    )(har, hbc, ea, cdr, C1, wr, c1, W2, b2, Ae, cae, Ae2)


def _node_body(h_ref, ag0_ref, ag1_ref, ac0_ref, ac1_ref,
               nh_ref, na_ref, cn1_ref, n2_ref, bn2_ref,
               ph_ref, cap_ref, ap2_ref, hnew_ref, acc_ref):
    h = h_ref[...]
    agg = ag0_ref[...] + ag1_ref[...]
    hn = _silu(_dot_t(h, nh_ref[...]) + _dot_t(agg, na_ref[...]) + cn1_ref[...])
    hn = _silu(_dot_t(hn, n2_ref[...]) + bn2_ref[...])
    h_new = h + hn
    p = _silu(_dot_t(h_new, ph_ref[...]) + cap_ref[...])
    ap = jnp.sum(_bf(p) * _bf(ap2_ref[...]), axis=1, keepdims=True)
    aggc = ac0_ref[...] + ac1_ref[...]
    hnew_ref[...] = h_new
    acc_ref[...] = aggc * ap


def _tc_node(h, ag0, ag1, ac0, ac1, Nh, Na, cn1, W_n2, b_n2, Ph, cap, Ap2):
    grid = (N_NODES // NBLK,)
    blk = pl.BlockSpec((NBLK, D), lambda i: (i, 0))
    cblk = pl.BlockSpec((NBLK, 4), lambda i: (i, 0))
    full = lambda shape: pl.BlockSpec(shape, lambda i: (0,) * len(shape))
    return pl.pallas_call(
        _node_body,
        grid=grid,
        in_specs=[blk, blk, blk, cblk, cblk,
                  full((D, D)), full((D, D)), full((1, D)),
                  full((D, D)), full((1, D)), full((D, D)), full((1, D)),
                  full((1, D))],
        out_specs=[blk, cblk],
        out_shape=[jax.ShapeDtypeStruct((N_NODES, D), jnp.float32),
                   jax.ShapeDtypeStruct((N_NODES, 4), jnp.float32)],
    )(h, ag0, ag1, ac0, ac1, Nh, Na, cn1, W_n2, b_n2, Ph, cap, Ap2)


CB = 128
NBIG = EPW // CB
CT = EPW - NBIG * CB


def _sc_gather_body(hA, hB, rowh, colh, cfl, outA, outB, outC,
                    ct_v, ri_v, ci_v, rA_v, rB_v, cdr_v,
                    ti_v, tj_v, tA_v, tB_v, tc_v,
                    semA, semB, semWA, semWB):
    cid = lax.axis_index("c")
    sid = lax.axis_index("s")
    wid = sid * NC + cid
    pltpu.sync_copy(cfl, ct_v)

    def coord_groups(riv, civ, cdrv, ngroups):
        for i in range(ngroups):
            ri3 = riv[pl.ds(i * 16, 16)] * 3
            ci3 = civ[pl.ds(i * 16, 16)] * 3
            lanes = lax.iota(jnp.int32, 16) + (i * 16)
            rad = None
            for k in range(3):
                a = plsc.load_gather(ct_v, [ri3 + k])
                b = plsc.load_gather(ct_v, [ci3 + k])
                dk = a - b
                plsc.store_scatter(cdrv, [lanes, jnp.full((16,), k, jnp.int32)], dk)
                rad = dk * dk if rad is None else rad + dk * dk
            plsc.store_scatter(cdrv, [lanes, jnp.full((16,), 3, jnp.int32)], rad)

    def do_chunk(base, n, riv, civ, rA, rB, cdrv):
        pltpu.sync_copy(rowh.at[pl.ds(base, n)], riv)
        pltpu.sync_copy(colh.at[pl.ds(base, n)], civ)
        gA = pltpu.async_copy(hA.at[riv], rA, semA)
        gB = pltpu.async_copy(hB.at[civ], rB, semB)
        gA.wait()
        wA = pltpu.async_copy(rA, outA.at[pl.ds(base, n)], semWA)
        gB.wait()
        wB = pltpu.async_copy(rB, outB.at[pl.ds(base, n)], semWB)
        coord_groups(riv, civ, cdrv, n // 16)
        pltpu.sync_copy(cdrv, outC.at[pl.ds(base, n)])
        wA.wait()
        wB.wait()

    def chunk(j, carry):
        do_chunk(wid * EPW + j * CB, CB, ri_v, ci_v, rA_v, rB_v, cdr_v)
        return carry

    lax.fori_loop(0, NBIG, chunk, 0)
    do_chunk(wid * EPW + NBIG * CB, CT, ti_v, tj_v, tA_v, tB_v, tc_v)


def _sc_gather(hA, hB, row, col, coord_flat):
    mesh = plsc.VectorSubcoreMesh(core_axis_name="c", subcore_axis_name="s")
    fn = pl.kernel(
        _sc_gather_body,
        out_type=[jax.ShapeDtypeStruct((N_EDGES, D), jnp.float32),
                  jax.ShapeDtypeStruct((N_EDGES, D), jnp.float32),
                  jax.ShapeDtypeStruct((N_EDGES, 4), jnp.float32)],
        mesh=mesh,
        scratch_types=[pltpu.VMEM((3 * N_NODES,), jnp.float32),
                       pltpu.VMEM((CB,), jnp.int32),
                       pltpu.VMEM((CB,), jnp.int32),
                       pltpu.VMEM((CB, D), jnp.float32),
                       pltpu.VMEM((CB, D), jnp.float32),
                       pltpu.VMEM((CB, 4), jnp.float32),
                       pltpu.VMEM((CT,), jnp.int32),
                       pltpu.VMEM((CT,), jnp.int32),
                       pltpu.VMEM((CT, D), jnp.float32),
                       pltpu.VMEM((CT, D), jnp.float32),
                       pltpu.VMEM((CT, 4), jnp.float32),
                       pltpu.SemaphoreType.DMA,
                       pltpu.SemaphoreType.DMA,
                       pltpu.SemaphoreType.DMA,
                       pltpu.SemaphoreType.DMA],
        compiler_params=pltpu.CompilerParams(needs_layout_passes=False),
    )
    return fn(hA, hB, row, col, coord_flat)


def _sc_scatter_body(vals, rowh, z128, out2, acc_sh, val_v, idx_v,
                     tval_v, tidx_v, semV, semI):
    cid = lax.axis_index("c")
    sid = lax.axis_index("s")
    pltpu.sync_copy(z128.at[pl.ds(sid * NSL, NSL)],
                    acc_sh.at[pl.ds(sid * NSL, NSL)])
    plsc.subcore_barrier()

    def do_chunk(base, n, vv, iv):
        cI = pltpu.async_copy(rowh.at[pl.ds(base, n)], iv, semI)
        cV = pltpu.async_copy(vals.at[pl.ds(base, n)], vv, semV)
        cI.wait()
        cV.wait()
        pltpu.sync_copy(vv, acc_sh.at[iv], add=True)

    def chunk(j, carry):
        do_chunk(cid * (N_EDGES // NC) + sid * EPW + j * CB, CB, val_v, idx_v)
        return carry

    lax.fori_loop(0, NBIG, chunk, 0)
    do_chunk(cid * (N_EDGES // NC) + sid * EPW + NBIG * CB, CT, tval_v, tidx_v)
    plsc.subcore_barrier()
    pltpu.sync_copy(acc_sh.at[pl.ds(sid * NSL, NSL)],
                    out2.at[pl.ds(cid * NP + sid * NSL, NSL)])


def _sc_scatter(vals, row, z128):
    mesh = plsc.VectorSubcoreMesh(core_axis_name="c", subcore_axis_name="s")
    fn = pl.kernel(
        _sc_scatter_body,
        out_type=[jax.ShapeDtypeStruct((2 * NP, D), jnp.float32)],
        mesh=mesh,
        scratch_types=[pltpu.VMEM_SHARED((NP, D), jnp.float32),
                       pltpu.VMEM((CB, D), jnp.float32),
                       pltpu.VMEM((CB,), jnp.int32),
                       pltpu.VMEM((CT, D), jnp.float32),
                       pltpu.VMEM((CT,), jnp.int32),
                       pltpu.SemaphoreType.DMA,
                       pltpu.SemaphoreType.DMA],
    )
    (out2,) = fn(vals, row, z128)
    return out2


def kernel(h, edge_index, coord, edge_attr, prompt, W_e1, b_e1, W_e2, b_e2,
           W_n1, b_n1, W_n2, b_n2, W_ae1, b_ae1, W_ae2, W_ap1, b_ap1, W_ap2):
    row = edge_index[0].astype(jnp.int32)
    col = edge_index[1].astype(jnp.int32)
    p0 = prompt[0]

    def _fold(w, b):
        return (jnp.matmul(w.astype(jnp.bfloat16), p0.astype(jnp.bfloat16),
                           preferred_element_type=jnp.float32) + b)[None, :]

    A1 = W_e1[:, 0:D]
    B1 = W_e1[:, D:2 * D]
    wr = W_e1[:, 2 * D].astype(jnp.bfloat16).astype(jnp.float32)[None, :]
    C1 = W_e1[:, 2 * D + 1:2 * D + 1 + DE]
    c1 = _fold(W_e1[:, 2 * D + 1 + DE:], b_e1)
    cae = _fold(W_ae1[:, D:], b_ae1)
    Ae = W_ae1[:, :D]
    cn1 = _fold(W_n1[:, 2 * D:], b_n1)
    Nh = W_n1[:, 0:D]
    Na = W_n1[:, D:2 * D]
    cap = _fold(W_ap1[:, D:], b_ap1)
    Ph = W_ap1[:, :D]

    hA, hB = _tc_pre(h, A1, B1)

    har, hbc, cdr = _sc_gather(hA, hB, row, col, coord.reshape(-1))

    ef, tr = _tc_edge(har, hbc, edge_attr, cdr, C1.T, wr, c1,
                      W_e2, b_e2[None, :], Ae, cae, W_ae2)

    z128 = jnp.zeros((NP, D), jnp.float32)
    agg2 = _sc_scatter(ef, row, z128)
    acc2 = _sc_scatter(tr, row, z128)
    ag0, ag1 = agg2[:N_NODES], agg2[NP:NP + N_NODES]
    ac0, ac1 = acc2[:N_NODES, :4], acc2[NP:NP + N_NODES, :4]

    h_new, acc4 = _tc_node(h, ag0, ag1, ac0, ac1, Nh, Na, cn1,
                           W_n2, b_n2[None, :], Ph, cap, W_ap2)
    return (h_new, coord, acc4[:, :3])


# speedup vs baseline: 3.6962x; 1.0298x over previous
import functools

import jax
import jax.numpy as jnp
from jax import lax
from jax.experimental import pallas as pl
from jax.experimental.pallas import tpu as pltpu
from jax.experimental.pallas import tpu_sc as plsc

N_NODES = 10000
N_EDGES = 320000
D = 128
DE = 16

NBLK = 1000
EBLK = 2000

NC = 2
NS = 16
NW = NC * NS
EPW = N_EDGES // NW
CG = 80
NP = 10112
NSL = NP // NS


def _silu(x):
    return x * jax.nn.sigmoid(x)


def _dot_t(x, w):
    return lax.dot_general(x.astype(jnp.bfloat16), w.astype(jnp.bfloat16),
                           (((1,), (1,)), ((), ())),
                           preferred_element_type=jnp.float32)


def _bf(x):
    return x.astype(jnp.bfloat16).astype(jnp.float32)


def _pre_body(h_ref, a1_ref, b1_ref, ha_ref, hb_ref):
    h = h_ref[...]
    ha_ref[...] = _dot_t(h, a1_ref[...])
    hb_ref[...] = _dot_t(h, b1_ref[...])


def _tc_pre(h, A1, B1):
    grid = (N_NODES // NBLK,)
    blk = pl.BlockSpec((NBLK, D), lambda i: (i, 0))
    wblk = pl.BlockSpec((D, D), lambda i: (0, 0))
    return pl.pallas_call(
        _pre_body,
        grid=grid,
        in_specs=[blk, wblk, wblk],
        out_specs=[blk, blk],
        out_shape=[jax.ShapeDtypeStruct((N_NODES, D), jnp.float32)] * 2,
    )(h, A1, B1)


def _edge_body(har_ref, hbc_ref, ea_ref, cdr_ref,
               c1t_ref, wr_ref, c1_ref, w2_ref, b2_ref, ae1_ref, cae_ref,
               ae2_ref, ef_ref, tr_ref):
    cdr = cdr_ref[...]
    rad = cdr[:, 3:4]
    ea = ea_ref[...]
    c1t = c1t_ref[...]
    t_ea = _bf(ea[:, 0:1]) * _bf(c1t[0:1, :])
    for k in range(1, DE):
        t_ea = t_ea + _bf(ea[:, k:k + 1]) * _bf(c1t[k:k + 1, :])
    t1 = (har_ref[...] + hbc_ref[...] + t_ea
          + _bf(rad) * wr_ref[...] + c1_ref[...])
    t1 = _silu(t1)
    ef = _silu(_dot_t(t1, w2_ref[...]) + b2_ref[...])
    g = _silu(_dot_t(ef, ae1_ref[...]) + cae_ref[...])
    ae = jnp.sum(_bf(g) * _bf(ae2_ref[...]), axis=1, keepdims=True)
    inv = 1.0 / jnp.maximum(jnp.sqrt(rad), 1e-12)
    ef_ref[...] = ef
    tr4 = cdr * (ae * inv)
    tr_ref[...] = jnp.concatenate(
        [tr4, jnp.zeros((tr4.shape[0], D - 4), jnp.float32)], axis=1)


def _tc_edge(har, hbc, ea, cdr, C1, wr, c1, W2, b2, Ae, cae, Ae2):
    grid = (N_EDGES // EBLK,)
    eblk = pl.BlockSpec((EBLK, D), lambda i: (i, 0))
    eablk = pl.BlockSpec((EBLK, DE), lambda i: (i, 0))
    cdblk = pl.BlockSpec((EBLK, 4), lambda i: (i, 0))
    full = lambda shape: pl.BlockSpec(shape, lambda i: (0,) * len(shape))
    return pl.pallas_call(
        _edge_body,
        grid=grid,
        in_specs=[eblk, eblk, eablk, cdblk,
                  full((DE, D)), full((1, D)), full((1, D)),
                  full((D, D)), full((1, D)), full((D, D)), full((1, D)),
                  full((1, D))],
        out_specs=[eblk, eblk],
        out_shape=[jax.ShapeDtypeStruct((N_EDGES, D), jnp.float32),
                   jax.ShapeDtypeStruct((N_EDGES, D), jnp.float32)],
    )(har, hbc, ea, cdr, C1, wr, c1, W2, b2, Ae, cae, Ae2)


def _node_body(h_ref, ag0_ref, ag1_ref, ac0_ref, ac1_ref,
               nh_ref, na_ref, cn1_ref, n2_ref, bn2_ref,
               ph_ref, cap_ref, ap2_ref, hnew_ref, acc_ref):
    h = h_ref[...]
    agg = ag0_ref[...] + ag1_ref[...]
    hn = _silu(_dot_t(h, nh_ref[...]) + _dot_t(agg, na_ref[...]) + cn1_ref[...])
    hn = _silu(_dot_t(hn, n2_ref[...]) + bn2_ref[...])
    h_new = h + hn
    p = _silu(_dot_t(h_new, ph_ref[...]) + cap_ref[...])
    ap = jnp.sum(_bf(p) * _bf(ap2_ref[...]), axis=1, keepdims=True)
    aggc = ac0_ref[...] + ac1_ref[...]
    hnew_ref[...] = h_new
    acc_ref[...] = aggc * ap


def _tc_node(h, ag0, ag1, ac0, ac1, Nh, Na, cn1, W_n2, b_n2, Ph, cap, Ap2):
    grid = (N_NODES // NBLK,)
    blk = pl.BlockSpec((NBLK, D), lambda i: (i, 0))
    cblk = pl.BlockSpec((NBLK, 4), lambda i: (i, 0))
    full = lambda shape: pl.BlockSpec(shape, lambda i: (0,) * len(shape))
    return pl.pallas_call(
        _node_body,
        grid=grid,
        in_specs=[blk, blk, blk, cblk, cblk,
                  full((D, D)), full((D, D)), full((1, D)),
                  full((D, D)), full((1, D)), full((D, D)), full((1, D)),
                  full((1, D))],
        out_specs=[blk, cblk],
        out_shape=[jax.ShapeDtypeStruct((N_NODES, D), jnp.float32),
                   jax.ShapeDtypeStruct((N_NODES, 4), jnp.float32)],
    )(h, ag0, ag1, ac0, ac1, Nh, Na, cn1, W_n2, b_n2, Ph, cap, Ap2)


CB = 128
NBIG = EPW // CB
CT = EPW - NBIG * CB


def _sc_gather_body(hA, hB, rowh, colh, cfl, outA, outB, outC,
                    ct_v, ri_v, ci_v, rA_v, rB_v, cdr_v,
                    ti_v, tj_v, tA_v, tB_v, tc_v,
                    semA, semB, semWA, semWB):
    cid = lax.axis_index("c")
    sid = lax.axis_index("s")
    wid = sid * NC + cid
    pltpu.sync_copy(cfl, ct_v)

    def coord_groups(riv, civ, cdrv, ngroups):
        for i in range(ngroups):
            ri3 = riv[pl.ds(i * 16, 16)] * 3
            ci3 = civ[pl.ds(i * 16, 16)] * 3
            lanes = lax.iota(jnp.int32, 16) + (i * 16)
            rad = None
            for k in range(3):
                a = plsc.load_gather(ct_v, [ri3 + k])
                b = plsc.load_gather(ct_v, [ci3 + k])
                dk = a - b
                plsc.store_scatter(cdrv, [lanes, jnp.full((16,), k, jnp.int32)], dk)
                rad = dk * dk if rad is None else rad + dk * dk
            plsc.store_scatter(cdrv, [lanes, jnp.full((16,), 3, jnp.int32)], rad)

    def do_chunk(base, n, riv, civ, rA, rB, cdrv):
        pltpu.sync_copy(rowh.at[pl.ds(base, n)], riv)
        pltpu.sync_copy(colh.at[pl.ds(base, n)], civ)
        gA = pltpu.async_copy(hA.at[riv], rA, semA)
        gB = pltpu.async_copy(hB.at[civ], rB, semB)
        gA.wait()
        wA = pltpu.async_copy(rA, outA.at[pl.ds(base, n)], semWA)
        gB.wait()
        wB = pltpu.async_copy(rB, outB.at[pl.ds(base, n)], semWB)
        coord_groups(riv, civ, cdrv, n // 16)
        pltpu.sync_copy(cdrv, outC.at[pl.ds(base, n)])
        wA.wait()
        wB.wait()

    def chunk(j, carry):
        do_chunk(wid * EPW + j * CB, CB, ri_v, ci_v, rA_v, rB_v, cdr_v)
        return carry

    lax.fori_loop(0, NBIG, chunk, 0)
    do_chunk(wid * EPW + NBIG * CB, CT, ti_v, tj_v, tA_v, tB_v, tc_v)


def _sc_gather(hA, hB, row, col, coord_flat):
    mesh = plsc.VectorSubcoreMesh(core_axis_name="c", subcore_axis_name="s")
    fn = pl.kernel(
        _sc_gather_body,
        out_type=[jax.ShapeDtypeStruct((N_EDGES, D), jnp.float32),
                  jax.ShapeDtypeStruct((N_EDGES, D), jnp.float32),
                  jax.ShapeDtypeStruct((N_EDGES, 4), jnp.float32)],
        mesh=mesh,
        scratch_types=[pltpu.VMEM((3 * N_NODES,), jnp.float32),
                       pltpu.VMEM((CB,), jnp.int32),
                       pltpu.VMEM((CB,), jnp.int32),
                       pltpu.VMEM((CB, D), jnp.float32),
                       pltpu.VMEM((CB, D), jnp.float32),
                       pltpu.VMEM((CB, 4), jnp.float32),
                       pltpu.VMEM((CT,), jnp.int32),
                       pltpu.VMEM((CT,), jnp.int32),
                       pltpu.VMEM((CT, D), jnp.float32),
                       pltpu.VMEM((CT, D), jnp.float32),
                       pltpu.VMEM((CT, 4), jnp.float32),
                       pltpu.SemaphoreType.DMA,
                       pltpu.SemaphoreType.DMA,
                       pltpu.SemaphoreType.DMA,
                       pltpu.SemaphoreType.DMA],
        compiler_params=pltpu.CompilerParams(needs_layout_passes=False),
    )
    return fn(hA, hB, row, col, coord_flat)


def _sc_scatter_body(vals, rowh, z128, out2, acc_sh, val_v, idx_v,
                     tval_v, tidx_v, semV, semI):
    cid = lax.axis_index("c")
    sid = lax.axis_index("s")
    pltpu.sync_copy(z128.at[pl.ds(sid * NSL, NSL)],
                    acc_sh.at[pl.ds(sid * NSL, NSL)])
    plsc.subcore_barrier()

    def do_chunk(base, n, vv, iv):
        cI = pltpu.async_copy(rowh.at[pl.ds(base, n)], iv, semI)
        cV = pltpu.async_copy(vals.at[pl.ds(base, n)], vv, semV)
        cI.wait()
        cV.wait()
        pltpu.sync_copy(vv, acc_sh.at[iv], add=True)

    def chunk(j, carry):
        do_chunk(cid * (N_EDGES // NC) + sid * EPW + j * CB, CB, val_v, idx_v)
        return carry

    lax.fori_loop(0, NBIG, chunk, 0)
    do_chunk(cid * (N_EDGES // NC) + sid * EPW + NBIG * CB, CT, tval_v, tidx_v)
    plsc.subcore_barrier()
    pltpu.sync_copy(acc_sh.at[pl.ds(sid * NSL, NSL)],
                    out2.at[pl.ds(cid * NP + sid * NSL, NSL)])


def _sc_scatter(vals, row, z128):
    mesh = plsc.VectorSubcoreMesh(core_axis_name="c", subcore_axis_name="s")
    fn = pl.kernel(
        _sc_scatter_body,
        out_type=[jax.ShapeDtypeStruct((2 * NP, D), jnp.float32)],
        mesh=mesh,
        scratch_types=[pltpu.VMEM_SHARED((NP, D), jnp.float32),
                       pltpu.VMEM((CB, D), jnp.float32),
                       pltpu.VMEM((CB,), jnp.int32),
                       pltpu.VMEM((CT, D), jnp.float32),
                       pltpu.VMEM((CT,), jnp.int32),
                       pltpu.SemaphoreType.DMA,
                       pltpu.SemaphoreType.DMA],
    )
    (out2,) = fn(vals, row, z128)
    return out2


def kernel(h, edge_index, coord, edge_attr, prompt, W_e1, b_e1, W_e2, b_e2,
           W_n1, b_n1, W_n2, b_n2, W_ae1, b_ae1, W_ae2, W_ap1, b_ap1, W_ap2):
    row = edge_index[0].astype(jnp.int32)
    col = edge_index[1].astype(jnp.int32)
    p0 = prompt[0]

    def _fold(w, b):
        return (jnp.matmul(w.astype(jnp.bfloat16), p0.astype(jnp.bfloat16),
                           preferred_element_type=jnp.float32) + b)[None, :]

    A1 = W_e1[:, 0:D]
    B1 = W_e1[:, D:2 * D]
    wr = W_e1[:, 2 * D].astype(jnp.bfloat16).astype(jnp.float32)[None, :]
    C1 = W_e1[:, 2 * D + 1:2 * D + 1 + DE]
    c1 = _fold(W_e1[:, 2 * D + 1 + DE:], b_e1)
    cae = _fold(W_ae1[:, D:], b_ae1)
    Ae = W_ae1[:, :D]
    cn1 = _fold(W_n1[:, 2 * D:], b_n1)
    Nh = W_n1[:, 0:D]
    Na = W_n1[:, D:2 * D]
    cap = _fold(W_ap1[:, D:], b_ap1)
    Ph = W_ap1[:, :D]

    hA, hB = _tc_pre(h, A1, B1)

    har, hbc, cdr = _sc_gather(hA, hB, row, col, coord.reshape(-1))

    ef, tr = _tc_edge(har, hbc, edge_attr, cdr, C1.T, wr, c1,
                      W_e2, b_e2[None, :], Ae, cae, W_ae2)

    z128 = jnp.zeros((NP, D), jnp.float32)
    agg2 = _sc_scatter(ef, row, z128)
    acc2 = _sc_scatter(tr, row, z128)
    ag0, ag1 = agg2[:N_NODES], agg2[NP:NP + N_NODES]
    ac0, ac1 = acc2[:N_NODES, :4], acc2[NP:NP + N_NODES, :4]

    h_new, acc4 = _tc_node(h, ag0, ag1, ac0, ac1, Nh, Na, cn1,
                           W_n2, b_n2[None, :], Ph, cap, W_ap2)
    return (h_new, coord, acc4[:, :3])
